# Initial kernel scaffold; baseline (speedup 1.0000x reference)
import jax
import jax.numpy as jnp
from jax.experimental import pallas as pl


def kernel(kv_cache, num_evicts, cachelens, n_local_heads, head_dim):
    raise NotImplementedError("write your pallas kernel here")


# speedup vs baseline: 8.3925x; 8.3925x over previous
import functools

import jax
import jax.numpy as jnp
from jax import lax
from jax.experimental import pallas as pl
from jax.experimental.pallas import tpu as pltpu
from jax.experimental.pallas import tpu_sc as plsc

_NUM_SINK = 4
_ROWS = 32768
_ROW_W = 1024
_NW = 32
_RPW = _ROWS // _NW
_CH = 32
_NCH = _RPW // _CH
_LANES = 16

_mesh = plsc.VectorSubcoreMesh(core_axis_name="c", subcore_axis_name="s")


@functools.partial(
    pl.kernel,
    out_type=jax.ShapeDtypeStruct((_ROWS, _ROW_W), jnp.float32),
    mesh=_mesh,
    compiler_params=pltpu.CompilerParams(needs_layout_passes=False),
    scratch_types=[
        pltpu.VMEM((_LANES,), jnp.int32),
        pltpu.VMEM((_LANES,), jnp.int32),
        pltpu.VMEM((_CH,), jnp.int32),
        pltpu.VMEM((_CH,), jnp.int32),
        pltpu.VMEM((_CH, _ROW_W), jnp.float32),
        pltpu.VMEM((_CH, _ROW_W), jnp.float32),
        pltpu.SemaphoreType.DMA,
        pltpu.SemaphoreType.DMA,
        pltpu.SemaphoreType.DMA,
        pltpu.SemaphoreType.DMA,
    ],
)
def _evict(cache_hbm, ne_hbm, cl_hbm, out_hbm,
           ne_v, cl_v, idx0, idx1, buf0, buf1,
           gsem0, gsem1, ssem0, ssem1):
    cid = lax.axis_index("c")
    sid = lax.axis_index("s")
    wid = sid * 2 + cid
    b = wid // 2
    base_row = wid * _RPW

    pltpu.sync_copy(ne_hbm, ne_v)
    pltpu.sync_copy(cl_hbm, cl_v)
    lanes = lax.iota(jnp.int32, _LANES)
    bvec = jnp.full((_LANES,), 0, jnp.int32) + b
    ne = plsc.load_gather(ne_v, [bvec])
    tg = plsc.load_gather(cl_v, [bvec]) - ne

    def fill_idx(c, idx_ref):
        row0 = base_row + c * _CH
        for k in range(_CH // _LANES):
            r = row0 + k * _LANES + lanes
            t = (((r >> 5) - (b << 6)) << 4) + (r & 15)
            st = t + jnp.where((t >= _NUM_SINK) & (t < tg), ne, 0)
            idx_ref[pl.ds(k * _LANES, _LANES)] = (
                (b << 11) + ((st >> 4) << 5) + (r & 16) + (st & 15)
            )

    def chunk(c, idx_ref, buf_ref, gsem, ssem):
        @pl.when(c >= 2)
        def _():
            pltpu.make_async_copy(
                buf_ref, out_hbm.at[pl.ds(base_row + (c - 2) * _CH, _CH)], ssem
            ).wait()
        fill_idx(c, idx_ref)
        pltpu.async_copy(cache_hbm.at[idx_ref], buf_ref, gsem).wait()
        pltpu.async_copy(
            buf_ref, out_hbm.at[pl.ds(base_row + c * _CH, _CH)], ssem
        )

    def outer(o, carry):
        chunk(2 * o, idx0, buf0, gsem0, ssem0)
        chunk(2 * o + 1, idx1, buf1, gsem1, ssem1)
        return carry

    lax.fori_loop(0, _NCH // 2, outer, 0)
    pltpu.make_async_copy(
        buf0, out_hbm.at[pl.ds(base_row + (_NCH - 2) * _CH, _CH)], ssem0
    ).wait()
    pltpu.make_async_copy(
        buf1, out_hbm.at[pl.ds(base_row + (_NCH - 1) * _CH, _CH)], ssem1
    ).wait()


def kernel(kv_cache, num_evicts, cachelens, n_local_heads, head_dim):
    flat = kv_cache.reshape(_ROWS, _ROW_W)
    out = _evict(flat,
                 num_evicts.astype(jnp.int32),
                 cachelens.astype(jnp.int32))
    return out.reshape(kv_cache.shape)


# speedup vs baseline: 22.5426x; 2.6860x over previous
import functools

import jax
import jax.numpy as jnp
from jax import lax
from jax.experimental import pallas as pl
from jax.experimental.pallas import tpu as pltpu
from jax.experimental.pallas import tpu_sc as plsc

_NUM_SINK = 4
_ROWS = 32768
_ROW_W = 1024
_NW = 32
_RPW = _ROWS // _NW
_CH = 32
_NCH = _RPW // _CH
_LANES = 16

_mesh = plsc.VectorSubcoreMesh(core_axis_name="c", subcore_axis_name="s")


@functools.partial(
    pl.kernel,
    out_type=jax.ShapeDtypeStruct((_ROWS, 8, 128), jnp.float32),
    mesh=_mesh,
    compiler_params=pltpu.CompilerParams(needs_layout_passes=False),
    scratch_types=[
        pltpu.VMEM((_LANES,), jnp.int32),
        pltpu.VMEM((_LANES,), jnp.int32),
        pltpu.VMEM((_CH,), jnp.int32),
        pltpu.VMEM((_CH,), jnp.int32),
        pltpu.VMEM((_CH, 8, 128), jnp.float32),
        pltpu.VMEM((_CH, 8, 128), jnp.float32),
        pltpu.SemaphoreType.DMA,
        pltpu.SemaphoreType.DMA,
        pltpu.SemaphoreType.DMA,
        pltpu.SemaphoreType.DMA,
    ],
)
def _evict(cache_hbm, ne_hbm, cl_hbm, out_hbm,
           ne_v, cl_v, idx0, idx1, buf0, buf1,
           gsem0, gsem1, ssem0, ssem1):
    cid = lax.axis_index("c")
    sid = lax.axis_index("s")
    wid = sid * 2 + cid
    b = wid // 2
    base_row = wid * _RPW

    pltpu.sync_copy(ne_hbm, ne_v)
    pltpu.sync_copy(cl_hbm, cl_v)
    lanes = lax.iota(jnp.int32, _LANES)
    bvec = jnp.full((_LANES,), 0, jnp.int32) + b
    ne = plsc.load_gather(ne_v, [bvec])
    tg = plsc.load_gather(cl_v, [bvec]) - ne

    def fill_idx(c, idx_ref):
        row0 = base_row + c * _CH
        for k in range(_CH // _LANES):
            r = row0 + k * _LANES + lanes
            t = (((r >> 5) - (b << 6)) << 4) + (r & 15)
            st = t + jnp.where((t >= _NUM_SINK) & (t < tg), ne, 0)
            idx_ref[pl.ds(k * _LANES, _LANES)] = (
                (b << 11) + ((st >> 4) << 5) + (r & 16) + (st & 15)
            )

    def chunk(c, idx_ref, buf_ref, gsem, ssem):
        @pl.when(c >= 2)
        def _():
            pltpu.make_async_copy(
                buf_ref, out_hbm.at[pl.ds(base_row + (c - 2) * _CH, _CH)], ssem
            ).wait()
        fill_idx(c, idx_ref)
        pltpu.async_copy(cache_hbm.at[idx_ref], buf_ref, gsem).wait()
        pltpu.async_copy(
            buf_ref, out_hbm.at[pl.ds(base_row + c * _CH, _CH)], ssem
        )

    def outer(o, carry):
        chunk(2 * o, idx0, buf0, gsem0, ssem0)
        chunk(2 * o + 1, idx1, buf1, gsem1, ssem1)
        return carry

    lax.fori_loop(0, _NCH // 2, outer, 0)
    pltpu.make_async_copy(
        buf0, out_hbm.at[pl.ds(base_row + (_NCH - 2) * _CH, _CH)], ssem0
    ).wait()
    pltpu.make_async_copy(
        buf1, out_hbm.at[pl.ds(base_row + (_NCH - 1) * _CH, _CH)], ssem1
    ).wait()


def kernel(kv_cache, num_evicts, cachelens, n_local_heads, head_dim):
    flat = kv_cache.reshape(_ROWS, 8, 128)
    out = _evict(flat,
                 num_evicts.astype(jnp.int32),
                 cachelens.astype(jnp.int32))
    return out.reshape(kv_cache.shape)


# speedup vs baseline: 22.6478x; 1.0047x over previous
import functools

import jax
import jax.numpy as jnp
from jax import lax
from jax.experimental import pallas as pl
from jax.experimental.pallas import tpu as pltpu
from jax.experimental.pallas import tpu_sc as plsc

_NUM_SINK = 4
_ROWS = 32768
_ROW_W = 1024
_NW = 32
_RPW = _ROWS // _NW
_CH = 32
_NCH = _RPW // _CH
_LANES = 16

_mesh = plsc.VectorSubcoreMesh(core_axis_name="c", subcore_axis_name="s")


@functools.partial(
    pl.kernel,
    out_type=jax.ShapeDtypeStruct((_ROWS, 8, 128), jnp.float32),
    mesh=_mesh,
    compiler_params=pltpu.CompilerParams(needs_layout_passes=False),
    scratch_types=[
        pltpu.VMEM((_LANES,), jnp.int32),
        pltpu.VMEM((_LANES,), jnp.int32),
        pltpu.VMEM((_CH,), jnp.int32),
        pltpu.VMEM((_CH,), jnp.int32),
        pltpu.VMEM((_CH,), jnp.int32),
        pltpu.VMEM((_CH, 8, 128), jnp.float32),
        pltpu.VMEM((_CH, 8, 128), jnp.float32),
        pltpu.VMEM((_CH, 8, 128), jnp.float32),
        pltpu.SemaphoreType.DMA,
        pltpu.SemaphoreType.DMA,
        pltpu.SemaphoreType.DMA,
        pltpu.SemaphoreType.DMA,
        pltpu.SemaphoreType.DMA,
        pltpu.SemaphoreType.DMA,
    ],
)
def _evict(cache_hbm, ne_hbm, cl_hbm, out_hbm,
           ne_v, cl_v, idx0, idx1, idx2, buf0, buf1, buf2,
           gsem0, gsem1, gsem2, ssem0, ssem1, ssem2):
    cid = lax.axis_index("c")
    sid = lax.axis_index("s")
    wid = sid * 2 + cid
    b = wid // 2
    base_row = wid * _RPW

    pltpu.sync_copy(ne_hbm, ne_v)
    pltpu.sync_copy(cl_hbm, cl_v)
    lanes = lax.iota(jnp.int32, _LANES)
    bvec = jnp.full((_LANES,), 0, jnp.int32) + b
    ne = plsc.load_gather(ne_v, [bvec])
    tg = plsc.load_gather(cl_v, [bvec]) - ne

    def fill_idx(c, idx_ref):
        row0 = base_row + c * _CH
        for k in range(_CH // _LANES):
            r = row0 + k * _LANES + lanes
            t = (((r >> 5) - (b << 6)) << 4) + (r & 15)
            st = t + jnp.where((t >= _NUM_SINK) & (t < tg), ne, 0)
            idx_ref[pl.ds(k * _LANES, _LANES)] = (
                (b << 11) + ((st >> 4) << 5) + (r & 16) + (st & 15)
            )

    idx = (idx0, idx1, idx2)
    buf = (buf0, buf1, buf2)
    gsem = (gsem0, gsem1, gsem2)
    ssem = (ssem0, ssem1, ssem2)

    def out_slice(c):
        return out_hbm.at[pl.ds(base_row + c * _CH, _CH)]

    def gstart(c, j):
        fill_idx(c, idx[j])
        pltpu.async_copy(cache_hbm.at[idx[j]], buf[j], gsem[j])

    gstart(0, 0)
    gstart(1, 1)

    def do_chunk(c, j, prefetch, drain):
        pltpu.make_async_copy(cache_hbm.at[idx[j]], buf[j], gsem[j]).wait()
        pltpu.async_copy(buf[j], out_slice(c), ssem[j])
        if prefetch:
            j2 = (j + 2) % 3
            if drain:
                pltpu.make_async_copy(buf[j2], out_slice(c - 1), ssem[j2]).wait()
            gstart(c + 2, j2)

    def outer(o, carry):
        c = 3 * o
        do_chunk(c, 0, True, True)
        do_chunk(c + 1, 1, True, True)
        do_chunk(c + 2, 2, True, True)
        return carry

    do_chunk(0, 0, True, False)
    do_chunk(1, 1, True, True)
    do_chunk(2, 2, True, True)
    lax.fori_loop(1, (_NCH - 2) // 3, outer, 0)
    do_chunk(_NCH - 2, (_NCH - 2) % 3, False, False)
    do_chunk(_NCH - 1, (_NCH - 1) % 3, False, False)
    for c in (_NCH - 3, _NCH - 2, _NCH - 1):
        pltpu.make_async_copy(buf[c % 3], out_slice(c), ssem[c % 3]).wait()


def kernel(kv_cache, num_evicts, cachelens, n_local_heads, head_dim):
    flat = kv_cache.reshape(_ROWS, 8, 128)
    out = _evict(flat,
                 num_evicts.astype(jnp.int32),
                 cachelens.astype(jnp.int32))
    return out.reshape(kv_cache.shape)


# speedup vs baseline: 22.6809x; 1.0015x over previous
import functools

import jax
import jax.numpy as jnp
from jax import lax
from jax.experimental import pallas as pl
from jax.experimental.pallas import tpu as pltpu
from jax.experimental.pallas import tpu_sc as plsc

_NUM_SINK = 4
_ROWS = 32768
_ROW_W = 1024
_NW = 32
_RPW = _ROWS // _NW
_CH = 32
_NCH = _RPW // _CH
_LANES = 16

_mesh = plsc.VectorSubcoreMesh(core_axis_name="c", subcore_axis_name="s")


@functools.partial(
    pl.kernel,
    out_type=jax.ShapeDtypeStruct((_ROWS, 8, 128), jnp.float32),
    mesh=_mesh,
    compiler_params=pltpu.CompilerParams(needs_layout_passes=False),
    scratch_types=[
        pltpu.VMEM((_LANES,), jnp.int32),
        pltpu.VMEM((_LANES,), jnp.int32),
        pltpu.VMEM((_CH,), jnp.int32),
        pltpu.VMEM((_CH,), jnp.int32),
        pltpu.VMEM((_CH,), jnp.int32),
        pltpu.VMEM((_CH, 8, 128), jnp.float32),
        pltpu.VMEM((_CH, 8, 128), jnp.float32),
        pltpu.VMEM((_CH, 8, 128), jnp.float32),
        pltpu.SemaphoreType.DMA,
        pltpu.SemaphoreType.DMA,
        pltpu.SemaphoreType.DMA,
        pltpu.SemaphoreType.DMA,
        pltpu.SemaphoreType.DMA,
        pltpu.SemaphoreType.DMA,
    ],
)
def _evict(cache_hbm, ne_hbm, cl_hbm, out_hbm,
           ne_v, cl_v, idx0, idx1, idx2, buf0, buf1, buf2,
           gsem0, gsem1, gsem2, ssem0, ssem1, ssem2):
    cid = lax.axis_index("c")
    sid = lax.axis_index("s")
    wid = sid * 2 + cid
    b = wid // 2
    base_row = wid * _RPW

    pltpu.sync_copy(ne_hbm, ne_v)
    pltpu.sync_copy(cl_hbm, cl_v)
    lanes = lax.iota(jnp.int32, _LANES)
    bvec = jnp.full((_LANES,), 0, jnp.int32) + b
    ne = plsc.load_gather(ne_v, [bvec])
    tg = plsc.load_gather(cl_v, [bvec]) - ne

    def fill_idx(c, idx_ref):
        row0 = base_row + c * _CH
        for k in range(_CH // _LANES):
            r = row0 + k * _LANES + lanes
            t = (((r >> 5) - (b << 6)) << 4) + (r & 15)
            st = t + jnp.where((t >= _NUM_SINK) & (t < tg), ne, 0)
            idx_ref[pl.ds(k * _LANES, _LANES)] = (
                (b << 11) + ((st >> 4) << 5) + (r & 16) + (st & 15)
            )

    idx = (idx0, idx1, idx2)
    buf = (buf0, buf1, buf2)
    gsem = (gsem0, gsem1, gsem2)
    ssem = (ssem0, ssem1, ssem2)

    def out_slice(c):
        return out_hbm.at[pl.ds(base_row + c * _CH, _CH)]

    def gstart(c, j):
        fill_idx(c, idx[j])
        pltpu.async_copy(cache_hbm.at[idx[j]], buf[j], gsem[j])

    gstart(0, 0)

    def do_chunk(c, j, prefetch, drain):
        pltpu.make_async_copy(cache_hbm.at[idx[j]], buf[j], gsem[j]).wait()
        pltpu.async_copy(buf[j], out_slice(c), ssem[j])
        if prefetch:
            j1 = (j + 1) % 3
            if drain:
                pltpu.make_async_copy(buf[j1], out_slice(c - 2), ssem[j1]).wait()
            gstart(c + 1, j1)

    def outer(o, carry):
        c = 3 * o
        do_chunk(c, 0, True, True)
        do_chunk(c + 1, 1, True, True)
        do_chunk(c + 2, 2, True, True)
        return carry

    do_chunk(0, 0, True, False)
    do_chunk(1, 1, True, False)
    do_chunk(2, 2, True, True)
    lax.fori_loop(1, (_NCH - 2) // 3, outer, 0)
    do_chunk(_NCH - 2, (_NCH - 2) % 3, True, True)
    do_chunk(_NCH - 1, (_NCH - 1) % 3, False, False)
    for c in (_NCH - 3, _NCH - 2, _NCH - 1):
        pltpu.make_async_copy(buf[c % 3], out_slice(c), ssem[c % 3]).wait()


def kernel(kv_cache, num_evicts, cachelens, n_local_heads, head_dim):
    flat = kv_cache.reshape(_ROWS, 8, 128)
    out = _evict(flat,
                 num_evicts.astype(jnp.int32),
                 cachelens.astype(jnp.int32))
    return out.reshape(kv_cache.shape)


# speedup vs baseline: 22.8469x; 1.0073x over previous
import functools

import jax
import jax.numpy as jnp
from jax import lax
from jax.experimental import pallas as pl
from jax.experimental.pallas import tpu as pltpu
from jax.experimental.pallas import tpu_sc as plsc

_NUM_SINK = 4
_ROWS = 32768
_ROW_W = 1024
_NW = 32
_RPW = _ROWS // _NW
_CH = 32
_NCH = _RPW // _CH
_LANES = 16

_mesh = plsc.VectorSubcoreMesh(core_axis_name="c", subcore_axis_name="s")


@functools.partial(
    pl.kernel,
    out_type=jax.ShapeDtypeStruct((_ROWS, 8, 128), jnp.float32),
    mesh=_mesh,
    compiler_params=pltpu.CompilerParams(needs_layout_passes=False),
    scratch_types=[
        pltpu.VMEM((_LANES,), jnp.int32),
        pltpu.VMEM((_LANES,), jnp.int32),
        pltpu.VMEM((_RPW,), jnp.int32),
        pltpu.VMEM((_CH, 8, 128), jnp.float32),
        pltpu.VMEM((_CH, 8, 128), jnp.float32),
        pltpu.VMEM((_CH, 8, 128), jnp.float32),
        pltpu.SemaphoreType.DMA,
        pltpu.SemaphoreType.DMA,
        pltpu.SemaphoreType.DMA,
        pltpu.SemaphoreType.DMA,
        pltpu.SemaphoreType.DMA,
        pltpu.SemaphoreType.DMA,
    ],
)
def _evict(cache_hbm, ne_hbm, cl_hbm, out_hbm,
           ne_v, cl_v, idx_all, buf0, buf1, buf2,
           gsem0, gsem1, gsem2, ssem0, ssem1, ssem2):
    cid = lax.axis_index("c")
    sid = lax.axis_index("s")
    wid = sid * 2 + cid
    b = wid // 2
    base_row = wid * _RPW

    pltpu.sync_copy(ne_hbm, ne_v)
    pltpu.sync_copy(cl_hbm, cl_v)
    lanes = lax.iota(jnp.int32, _LANES)
    bvec = jnp.full((_LANES,), 0, jnp.int32) + b
    ne = plsc.load_gather(ne_v, [bvec])
    tg = plsc.load_gather(cl_v, [bvec]) - ne

    def fill_group(k, carry):
        r = base_row + k * _LANES + lanes
        t = (((r >> 5) - (b << 6)) << 4) + (r & 15)
        st = t + jnp.where((t >= _NUM_SINK) & (t < tg), ne, 0)
        idx_all[pl.ds(k * _LANES, _LANES)] = (
            (b << 11) + ((st >> 4) << 5) + (r & 16) + (st & 15)
        )
        return carry

    lax.fori_loop(0, _RPW // _LANES, fill_group, 0)

    buf = (buf0, buf1, buf2)
    gsem = (gsem0, gsem1, gsem2)
    ssem = (ssem0, ssem1, ssem2)

    def out_slice(c):
        return out_hbm.at[pl.ds(base_row + c * _CH, _CH)]

    def idx_slice(c):
        return idx_all.at[pl.ds(c * _CH, _CH)]

    def gstart(c, j):
        pltpu.async_copy(cache_hbm.at[idx_slice(c)], buf[j], gsem[j])

    gstart(0, 0)

    def do_chunk(c, j, prefetch, drain):
        pltpu.make_async_copy(cache_hbm.at[idx_slice(c)], buf[j], gsem[j]).wait()
        pltpu.async_copy(buf[j], out_slice(c), ssem[j])
        if prefetch:
            j1 = (j + 1) % 3
            if drain:
                pltpu.make_async_copy(buf[j1], out_slice(c - 2), ssem[j1]).wait()
            gstart(c + 1, j1)

    def outer(o, carry):
        c = 3 * o
        do_chunk(c, 0, True, True)
        do_chunk(c + 1, 1, True, True)
        do_chunk(c + 2, 2, True, True)
        return carry

    do_chunk(0, 0, True, False)
    do_chunk(1, 1, True, False)
    do_chunk(2, 2, True, True)
    lax.fori_loop(1, (_NCH - 2) // 3, outer, 0)
    do_chunk(_NCH - 2, (_NCH - 2) % 3, True, True)
    do_chunk(_NCH - 1, (_NCH - 1) % 3, False, False)
    for c in (_NCH - 3, _NCH - 2, _NCH - 1):
        pltpu.make_async_copy(buf[c % 3], out_slice(c), ssem[c % 3]).wait()


def kernel(kv_cache, num_evicts, cachelens, n_local_heads, head_dim):
    flat = kv_cache.reshape(_ROWS, 8, 128)
    out = _evict(flat,
                 num_evicts.astype(jnp.int32),
                 cachelens.astype(jnp.int32))
    return out.reshape(kv_cache.shape)
